# Initial kernel scaffold; baseline (speedup 1.0000x reference)
import jax
import jax.numpy as jnp
from jax.experimental import pallas as pl


def kernel(x, edge_index, W1, b1, W2, b2, Wmu, bmu, Wls, bls):
    raise NotImplementedError("write your pallas kernel here")


# speedup vs baseline: 5.0524x; 5.0524x over previous
import functools

import jax
import jax.numpy as jnp
from jax import lax
from jax.experimental import pallas as pl
from jax.experimental.pallas import tpu as pltpu
from jax.experimental.pallas import tpu_sc as plsc

N = 10000
NP = 10240
E = 320000
K = 128
ERP = 2560
EP = ERP * K
RT32 = ERP // 32
RT16 = ERP // 16
TROWS = NP // 16
BR = 1024
GN = NP // BR
BRE = 2048
GE = EP // BRE

_MESH = plsc.VectorSubcoreMesh(core_axis_name="c", subcore_axis_name="s")


def _deg(dstp):
    @functools.partial(
        pl.kernel,
        out_type=jax.ShapeDtypeStruct((2 * NP,), jnp.float32),
        mesh=_MESH,
        compiler_params=pltpu.CompilerParams(use_tc_tiling_on_sc=False),
        scratch_types=[
            pltpu.VMEM_SHARED((NP,), jnp.float32),
            pltpu.VMEM((RT32, K), jnp.int32),
            pltpu.VMEM((K,), jnp.float32),
            pltpu.VMEM((TROWS,), jnp.float32),
        ],
    )
    def deg_kernel(dst_hbm, out_hbm, acc, idxd, ones_v, zer_v):
        c = lax.axis_index("c")
        s = lax.axis_index("s")
        w = s * 2 + c
        for i in range(K // 16):
            ones_v[pl.ds(16 * i, 16)] = jnp.ones((16,), jnp.float32)
        for i in range(TROWS // 16):
            zer_v[pl.ds(16 * i, 16)] = jnp.zeros((16,), jnp.float32)
        pltpu.sync_copy(zer_v, acc.at[pl.ds(s * TROWS, TROWS)])
        pltpu.sync_copy(dst_hbm.at[pl.ds(w * RT32, RT32)], idxd)
        plsc.subcore_barrier()

        def body(j, carry):
            pltpu.sync_copy(ones_v, acc.at[idxd.at[j]], add=True)
            return carry

        lax.fori_loop(0, RT32, body, 0)
        plsc.subcore_barrier()
        pltpu.sync_copy(acc.at[pl.ds(s * TROWS, TROWS)],
                        out_hbm.at[pl.ds(c * NP + s * TROWS, TROWS)])

    return deg_kernel(dstp)


def _agg(tbl, srcp, dstp):
    @functools.partial(
        pl.kernel,
        out_type=jax.ShapeDtypeStruct((2 * NP, 64), jnp.float32),
        mesh=_MESH,
        compiler_params=pltpu.CompilerParams(use_tc_tiling_on_sc=False),
        scratch_types=[
            pltpu.VMEM_SHARED((NP, 64), jnp.float32),
            pltpu.VMEM((RT16, K), jnp.int32),
            pltpu.VMEM((RT16, K), jnp.int32),
            pltpu.VMEM((K, 64), jnp.float32),
            pltpu.SemaphoreType.DMA,
        ],
    )
    def agg_kernel(tbl_hbm, src_hbm, dst_hbm, out_hbm, acc, idxs, idxd, rows,
                   sem):
        c = lax.axis_index("c")
        s = lax.axis_index("s")
        rb = s * TROWS
        pltpu.sync_copy(tbl_hbm.at[pl.ds(c * NP + rb, TROWS)],
                        acc.at[pl.ds(rb, TROWS)])
        pltpu.sync_copy(src_hbm.at[pl.ds(s * RT16, RT16)], idxs)
        pltpu.sync_copy(dst_hbm.at[pl.ds(s * RT16, RT16)], idxd)
        off = c * NP

        def addoff(j, carry):
            for i in range(K // 16):
                idxs[j, pl.ds(16 * i, 16)] = idxs[j, pl.ds(16 * i, 16)] + off
            return carry

        lax.fori_loop(0, RT16, addoff, 0)
        plsc.subcore_barrier()

        def body(j, carry):
            pltpu.async_copy(tbl_hbm.at[idxs.at[j]], rows, sem).wait()
            pltpu.sync_copy(rows, acc.at[idxd.at[j]], add=True)
            return carry

        lax.fori_loop(0, RT16, body, 0)
        plsc.subcore_barrier()
        pltpu.sync_copy(acc.at[pl.ds(rb, TROWS)],
                        out_hbm.at[pl.ds(c * NP + rb, TROWS)])

    return agg_kernel(tbl, srcp, dstp)


def _dec(mu, srcp, dstp):
    @functools.partial(
        pl.kernel,
        out_type=[jax.ShapeDtypeStruct((EP, 64), jnp.float32),
                  jax.ShapeDtypeStruct((EP, 64), jnp.float32)],
        mesh=_MESH,
        compiler_params=pltpu.CompilerParams(use_tc_tiling_on_sc=False),
        scratch_types=[
            pltpu.VMEM((RT32, K), jnp.int32),
            pltpu.VMEM((RT32, K), jnp.int32),
            pltpu.VMEM((K, 64), jnp.float32),
            pltpu.VMEM((K, 64), jnp.float32),
            pltpu.SemaphoreType.DMA,
            pltpu.SemaphoreType.DMA,
        ],
    )
    def dec_kernel(mu_hbm, src_hbm, dst_hbm, zs_hbm, zd_hbm, idxs, idxd,
                   zsv, zdv, sems, semd):
        c = lax.axis_index("c")
        s = lax.axis_index("s")
        w = s * 2 + c
        pltpu.sync_copy(src_hbm.at[pl.ds(w * RT32, RT32)], idxs)
        pltpu.sync_copy(dst_hbm.at[pl.ds(w * RT32, RT32)], idxd)

        def body(j, carry):
            cs = pltpu.async_copy(mu_hbm.at[idxs.at[j]], zsv, sems)
            cd = pltpu.async_copy(mu_hbm.at[idxd.at[j]], zdv, semd)
            cs.wait()
            cd.wait()
            eb = (w * RT32 + j) * K
            pltpu.sync_copy(zsv, zs_hbm.at[pl.ds(eb, K)])
            pltpu.sync_copy(zdv, zd_hbm.at[pl.ds(eb, K)])
            return carry

        lax.fori_loop(0, RT32, body, 0)

    return dec_kernel(mu, srcp, dstp)


def _tc_prep(xp, w1, degp):
    def body(x_ref, w_ref, d0_ref, d1_ref, dis_ref, o_ref):
        deg = d0_ref[0] + d1_ref[0] + 1.0
        dis = lax.rsqrt(deg)
        dis_ref[...] = dis
        h = jnp.dot(x_ref[...], w_ref[...],
                    preferred_element_type=jnp.float32) * dis
        o_ref[0] = h[:, :64]
        o_ref[1] = h[:, 64:]

    return pl.pallas_call(
        body,
        grid=(GN,),
        in_specs=[
            pl.BlockSpec((BR, 128), lambda i: (i, 0)),
            pl.BlockSpec((128, 128), lambda i: (0, 0)),
            pl.BlockSpec((1, BR, 1), lambda i: (0, i, 0)),
            pl.BlockSpec((1, BR, 1), lambda i: (1, i, 0)),
        ],
        out_specs=[
            pl.BlockSpec((BR, 1), lambda i: (i, 0)),
            pl.BlockSpec((2, BR, 64), lambda i: (0, i, 0)),
        ],
        out_shape=[
            jax.ShapeDtypeStruct((NP, 1), jnp.float32),
            jax.ShapeDtypeStruct((2, NP, 64), jnp.float32),
        ],
    )(xp, w1, degp, degp)


def _tc_mid1(p, b, dis, w):
    def body(p_ref, b_ref, dis_ref, w_ref, oa_ref, ob_ref):
        agg = jnp.concatenate([p_ref[0], p_ref[1]], axis=1)
        dis_v = dis_ref[...]
        h = jnp.maximum(agg * dis_v + b_ref[...], 0.0)
        v = jnp.dot(h, w_ref[...], preferred_element_type=jnp.float32) * dis_v
        oa_ref[0] = v[:, 0:64]
        oa_ref[1] = v[:, 128:192]
        ob_ref[0] = v[:, 64:128]
        ob_ref[1] = v[:, 192:256]

    return pl.pallas_call(
        body,
        grid=(GN,),
        in_specs=[
            pl.BlockSpec((2, BR, 64), lambda i: (0, i, 0)),
            pl.BlockSpec((1, 128), lambda i: (0, 0)),
            pl.BlockSpec((BR, 1), lambda i: (i, 0)),
            pl.BlockSpec((128, 256), lambda i: (0, 0)),
        ],
        out_specs=[
            pl.BlockSpec((2, BR, 64), lambda i: (0, i, 0)),
            pl.BlockSpec((2, BR, 64), lambda i: (0, i, 0)),
        ],
        out_shape=[
            jax.ShapeDtypeStruct((2, NP, 64), jnp.float32),
            jax.ShapeDtypeStruct((2, NP, 64), jnp.float32),
        ],
    )(p, b, dis, w)


def _tc_mid2(pa, pb, b, dis, w):
    def body(pa_ref, pb_ref, b_ref, dis_ref, w_ref, o_ref):
        agg = jnp.concatenate(
            [pa_ref[0], pb_ref[0], pa_ref[1], pb_ref[1]], axis=1)
        dis_v = dis_ref[...]
        h = jnp.maximum(agg * dis_v + b_ref[...], 0.0)
        v = jnp.dot(h, w_ref[...], preferred_element_type=jnp.float32) * dis_v
        o_ref[0] = v[:, :64]
        o_ref[1] = v[:, 64:]

    return pl.pallas_call(
        body,
        grid=(GN,),
        in_specs=[
            pl.BlockSpec((2, BR, 64), lambda i: (0, i, 0)),
            pl.BlockSpec((2, BR, 64), lambda i: (0, i, 0)),
            pl.BlockSpec((1, 256), lambda i: (0, 0)),
            pl.BlockSpec((BR, 1), lambda i: (i, 0)),
            pl.BlockSpec((256, 128), lambda i: (0, 0)),
        ],
        out_specs=pl.BlockSpec((2, BR, 64), lambda i: (0, i, 0)),
        out_shape=jax.ShapeDtypeStruct((2, NP, 64), jnp.float32),
    )(pa, pb, b, dis, w)


def _tc_fin(p, bmu, bls, dis):
    def body(p_ref, bmu_ref, bls_ref, dis_ref, mu_ref, ls_ref):
        dis_v = dis_ref[...]
        mu_ref[...] = p_ref[0] * dis_v + bmu_ref[...]
        ls_ref[...] = p_ref[1] * dis_v + bls_ref[...]

    return pl.pallas_call(
        body,
        grid=(GN,),
        in_specs=[
            pl.BlockSpec((2, BR, 64), lambda i: (0, i, 0)),
            pl.BlockSpec((1, 64), lambda i: (0, 0)),
            pl.BlockSpec((1, 64), lambda i: (0, 0)),
            pl.BlockSpec((BR, 1), lambda i: (i, 0)),
        ],
        out_specs=[
            pl.BlockSpec((BR, 64), lambda i: (i, 0)),
            pl.BlockSpec((BR, 64), lambda i: (i, 0)),
        ],
        out_shape=[
            jax.ShapeDtypeStruct((NP, 64), jnp.float32),
            jax.ShapeDtypeStruct((NP, 64), jnp.float32),
        ],
    )(p, bmu, bls, dis)


def _tc_dot(zs, zd):
    def body(zs_ref, zd_ref, o_ref):
        v = jnp.sum(zs_ref[...] * zd_ref[...], axis=1, keepdims=True)
        o_ref[...] = 1.0 / (1.0 + jnp.exp(-v))

    return pl.pallas_call(
        body,
        grid=(GE,),
        in_specs=[
            pl.BlockSpec((BRE, 64), lambda i: (i, 0)),
            pl.BlockSpec((BRE, 64), lambda i: (i, 0)),
        ],
        out_specs=pl.BlockSpec((BRE, 1), lambda i: (i, 0)),
        out_shape=jax.ShapeDtypeStruct((EP, 1), jnp.float32),
    )(zs, zd)


def kernel(x, edge_index, W1, b1, W2, b2, Wmu, bmu, Wls, bls):
    xp = jnp.pad(x, ((0, NP - N), (0, 0)))
    pad = jnp.full((EP - E,), N, jnp.int32)
    srcp = jnp.concatenate([edge_index[0], pad]).reshape(ERP, K)
    dstp = jnp.concatenate([edge_index[1], pad]).reshape(ERP, K)
    w3 = jnp.concatenate([Wmu, Wls], axis=1)

    degp = _deg(dstp).reshape(2, NP, 1)
    dis, v1 = _tc_prep(xp, W1, degp)
    p1 = _agg(v1.reshape(2 * NP, 64), srcp, dstp)
    v2a, v2b = _tc_mid1(p1.reshape(2, NP, 64), b1.reshape(1, 128), dis, W2)
    p2a = _agg(v2a.reshape(2 * NP, 64), srcp, dstp)
    p2b = _agg(v2b.reshape(2 * NP, 64), srcp, dstp)
    v3 = _tc_mid2(p2a.reshape(2, NP, 64), p2b.reshape(2, NP, 64),
                  b2.reshape(1, 256), dis, w3)
    p3 = _agg(v3.reshape(2 * NP, 64), srcp, dstp)
    mu, logstd = _tc_fin(p3.reshape(2, NP, 64), bmu.reshape(1, 64),
                         bls.reshape(1, 64), dis)
    zs, zd = _dec(mu, srcp, dstp)
    recon = _tc_dot(zs, zd)
    return (recon.reshape(-1)[:E], mu[:N], logstd[:N])


# speedup vs baseline: 6.0606x; 1.1995x over previous
import functools

import jax
import jax.numpy as jnp
from jax import lax
from jax.experimental import pallas as pl
from jax.experimental.pallas import tpu as pltpu
from jax.experimental.pallas import tpu_sc as plsc

N = 10000
NP = 10240
E = 320000
K = 128
ERP = 2560
EP = ERP * K
RT32 = ERP // 32
RT16 = ERP // 16
TROWS = NP // 16
BR = 1024
GN = NP // BR
BRE = 2048
GE = EP // BRE
RNG = 4
LK = 2

_MESH = plsc.VectorSubcoreMesh(core_axis_name="c", subcore_axis_name="s")


def _deg(dstp):
    @functools.partial(
        pl.kernel,
        out_type=jax.ShapeDtypeStruct((2 * NP,), jnp.float32),
        mesh=_MESH,
        compiler_params=pltpu.CompilerParams(use_tc_tiling_on_sc=False),
        scratch_types=[
            pltpu.VMEM_SHARED((NP,), jnp.float32),
            pltpu.VMEM((RT32, K), jnp.int32),
            pltpu.VMEM((K,), jnp.float32),
            pltpu.VMEM((TROWS,), jnp.float32),
            pltpu.SemaphoreType.DMA,
        ],
    )
    def deg_kernel(dst_hbm, out_hbm, acc, idxd, ones_v, zer_v, sem):
        c = lax.axis_index("c")
        s = lax.axis_index("s")
        w = s * 2 + c
        for i in range(K // 16):
            ones_v[pl.ds(16 * i, 16)] = jnp.ones((16,), jnp.float32)
        for i in range(TROWS // 16):
            zer_v[pl.ds(16 * i, 16)] = jnp.zeros((16,), jnp.float32)
        pltpu.sync_copy(zer_v, acc.at[pl.ds(s * TROWS, TROWS)])
        pltpu.sync_copy(dst_hbm.at[pl.ds(w * RT32, RT32)], idxd)
        plsc.subcore_barrier()

        def body(i, carry):
            for b in range(8):
                pltpu.async_copy(ones_v, acc.at[idxd.at[i * 8 + b]], sem,
                                 add=True)
            for b in range(8):
                pltpu.make_async_copy(ones_v, acc.at[idxd.at[i * 8 + b]],
                                      sem).wait()
            return carry

        lax.fori_loop(0, RT32 // 8, body, 0)
        plsc.subcore_barrier()
        pltpu.sync_copy(acc.at[pl.ds(s * TROWS, TROWS)],
                        out_hbm.at[pl.ds(c * NP + s * TROWS, TROWS)])

    return deg_kernel(dstp)


def _agg(tbl, srcp, dstp):
    @functools.partial(
        pl.kernel,
        out_type=jax.ShapeDtypeStruct((2 * NP, 64), jnp.float32),
        mesh=_MESH,
        compiler_params=pltpu.CompilerParams(use_tc_tiling_on_sc=False),
        scratch_types=[
            pltpu.VMEM_SHARED((NP, 64), jnp.float32),
            pltpu.VMEM((RT16, K), jnp.int32),
            pltpu.VMEM((RT16, K), jnp.int32),
            pltpu.VMEM((RNG, K, 64), jnp.float32),
            pltpu.SemaphoreType.DMA((RNG,)),
            pltpu.SemaphoreType.DMA((RNG,)),
        ],
    )
    def agg_kernel(tbl_hbm, src_hbm, dst_hbm, out_hbm, acc, idxs, idxd, rows,
                   gsem, ssem):
        c = lax.axis_index("c")
        s = lax.axis_index("s")
        rb = s * TROWS
        pltpu.sync_copy(tbl_hbm.at[pl.ds(c * NP + rb, TROWS)],
                        acc.at[pl.ds(rb, TROWS)])
        pltpu.sync_copy(src_hbm.at[pl.ds(s * RT16, RT16)], idxs)
        pltpu.sync_copy(dst_hbm.at[pl.ds(s * RT16, RT16)], idxd)
        off = c * NP

        def addoff(j, carry):
            for i in range(K // 16):
                idxs[j, pl.ds(16 * i, 16)] = idxs[j, pl.ds(16 * i, 16)] + off
            return carry

        lax.fori_loop(0, RT16, addoff, 0)
        plsc.subcore_barrier()

        def fire_g(j, r):
            pltpu.async_copy(tbl_hbm.at[idxs.at[j]], rows.at[r], gsem.at[r])

        def wait_g(j, r):
            pltpu.make_async_copy(tbl_hbm.at[idxs.at[j]], rows.at[r],
                                  gsem.at[r]).wait()

        def fire_s(j, r):
            pltpu.async_copy(rows.at[r], acc.at[idxd.at[j]], ssem.at[r],
                             add=True)

        def wait_s(j, r):
            pltpu.make_async_copy(rows.at[r], acc.at[idxd.at[j]],
                                  ssem.at[r]).wait()

        for r in range(LK):
            fire_g(r, r)

        def body(i, carry):
            for r in range(RNG):
                j = i * RNG + r
                rl = (r + LK) % RNG

                @pl.when(j + LK < RT16)
                def _():
                    @pl.when(j + LK >= RNG)
                    def _():
                        wait_s(j + LK - RNG, rl)
                    fire_g(j + LK, rl)

                wait_g(j, r)
                fire_s(j, r)
            return carry

        lax.fori_loop(0, RT16 // RNG, body, 0)
        for r in range(RNG):
            wait_s(RT16 - RNG + r, r)
        plsc.subcore_barrier()
        pltpu.sync_copy(acc.at[pl.ds(rb, TROWS)],
                        out_hbm.at[pl.ds(c * NP + rb, TROWS)])

    return agg_kernel(tbl, srcp, dstp)


def _dec(mu, srcp, dstp):
    @functools.partial(
        pl.kernel,
        out_type=[jax.ShapeDtypeStruct((EP, 64), jnp.float32),
                  jax.ShapeDtypeStruct((EP, 64), jnp.float32)],
        mesh=_MESH,
        compiler_params=pltpu.CompilerParams(use_tc_tiling_on_sc=False),
        scratch_types=[
            pltpu.VMEM((RT32, K), jnp.int32),
            pltpu.VMEM((RT32, K), jnp.int32),
            pltpu.VMEM((RNG, K, 64), jnp.float32),
            pltpu.VMEM((RNG, K, 64), jnp.float32),
            pltpu.SemaphoreType.DMA((RNG,)),
            pltpu.SemaphoreType.DMA((RNG,)),
            pltpu.SemaphoreType.DMA((RNG,)),
            pltpu.SemaphoreType.DMA((RNG,)),
        ],
    )
    def dec_kernel(mu_hbm, src_hbm, dst_hbm, zs_hbm, zd_hbm, idxs, idxd,
                   zsv, zdv, gs_sem, gd_sem, ws_sem, wd_sem):
        c = lax.axis_index("c")
        s = lax.axis_index("s")
        w = s * 2 + c
        pltpu.sync_copy(src_hbm.at[pl.ds(w * RT32, RT32)], idxs)
        pltpu.sync_copy(dst_hbm.at[pl.ds(w * RT32, RT32)], idxd)

        def fire_g(j, r):
            pltpu.async_copy(mu_hbm.at[idxs.at[j]], zsv.at[r], gs_sem.at[r])
            pltpu.async_copy(mu_hbm.at[idxd.at[j]], zdv.at[r], gd_sem.at[r])

        def wait_g(j, r):
            pltpu.make_async_copy(mu_hbm.at[idxs.at[j]], zsv.at[r],
                                  gs_sem.at[r]).wait()
            pltpu.make_async_copy(mu_hbm.at[idxd.at[j]], zdv.at[r],
                                  gd_sem.at[r]).wait()

        def fire_w(j, r):
            eb = (w * RT32 + j) * K
            pltpu.async_copy(zsv.at[r], zs_hbm.at[pl.ds(eb, K)], ws_sem.at[r])
            pltpu.async_copy(zdv.at[r], zd_hbm.at[pl.ds(eb, K)], wd_sem.at[r])

        def wait_w(j, r):
            eb = (w * RT32 + j) * K
            pltpu.make_async_copy(zsv.at[r], zs_hbm.at[pl.ds(eb, K)],
                                  ws_sem.at[r]).wait()
            pltpu.make_async_copy(zdv.at[r], zd_hbm.at[pl.ds(eb, K)],
                                  wd_sem.at[r]).wait()

        for r in range(LK):
            fire_g(r, r)

        def body(i, carry):
            for r in range(RNG):
                j = i * RNG + r
                rl = (r + LK) % RNG

                @pl.when(j + LK < RT32)
                def _():
                    @pl.when(j + LK >= RNG)
                    def _():
                        wait_w(j + LK - RNG, rl)
                    fire_g(j + LK, rl)

                wait_g(j, r)
                fire_w(j, r)
            return carry

        lax.fori_loop(0, RT32 // RNG, body, 0)
        for r in range(RNG):
            wait_w(RT32 - RNG + r, r)

    return dec_kernel(mu, srcp, dstp)


def _tc_prep(xp, w1, degp):
    def body(x_ref, w_ref, d0_ref, d1_ref, dis_ref, o_ref):
        deg = d0_ref[0] + d1_ref[0] + 1.0
        dis = lax.rsqrt(deg)
        dis_ref[...] = dis
        h = jnp.dot(x_ref[...], w_ref[...],
                    preferred_element_type=jnp.float32) * dis
        o_ref[0] = h[:, :64]
        o_ref[1] = h[:, 64:]

    return pl.pallas_call(
        body,
        grid=(GN,),
        in_specs=[
            pl.BlockSpec((BR, 128), lambda i: (i, 0)),
            pl.BlockSpec((128, 128), lambda i: (0, 0)),
            pl.BlockSpec((1, BR, 1), lambda i: (0, i, 0)),
            pl.BlockSpec((1, BR, 1), lambda i: (1, i, 0)),
        ],
        out_specs=[
            pl.BlockSpec((BR, 1), lambda i: (i, 0)),
            pl.BlockSpec((2, BR, 64), lambda i: (0, i, 0)),
        ],
        out_shape=[
            jax.ShapeDtypeStruct((NP, 1), jnp.float32),
            jax.ShapeDtypeStruct((2, NP, 64), jnp.float32),
        ],
    )(xp, w1, degp, degp)


def _tc_mid1(p, b, dis, w):
    def body(p_ref, b_ref, dis_ref, w_ref, oa_ref, ob_ref):
        agg = jnp.concatenate([p_ref[0], p_ref[1]], axis=1)
        dis_v = dis_ref[...]
        h = jnp.maximum(agg * dis_v + b_ref[...], 0.0)
        v = jnp.dot(h, w_ref[...], preferred_element_type=jnp.float32) * dis_v
        oa_ref[0] = v[:, 0:64]
        oa_ref[1] = v[:, 128:192]
        ob_ref[0] = v[:, 64:128]
        ob_ref[1] = v[:, 192:256]

    return pl.pallas_call(
        body,
        grid=(GN,),
        in_specs=[
            pl.BlockSpec((2, BR, 64), lambda i: (0, i, 0)),
            pl.BlockSpec((1, 128), lambda i: (0, 0)),
            pl.BlockSpec((BR, 1), lambda i: (i, 0)),
            pl.BlockSpec((128, 256), lambda i: (0, 0)),
        ],
        out_specs=[
            pl.BlockSpec((2, BR, 64), lambda i: (0, i, 0)),
            pl.BlockSpec((2, BR, 64), lambda i: (0, i, 0)),
        ],
        out_shape=[
            jax.ShapeDtypeStruct((2, NP, 64), jnp.float32),
            jax.ShapeDtypeStruct((2, NP, 64), jnp.float32),
        ],
    )(p, b, dis, w)


def _tc_mid2(pa, pb, b, dis, w):
    def body(pa_ref, pb_ref, b_ref, dis_ref, w_ref, o_ref):
        agg = jnp.concatenate(
            [pa_ref[0], pb_ref[0], pa_ref[1], pb_ref[1]], axis=1)
        dis_v = dis_ref[...]
        h = jnp.maximum(agg * dis_v + b_ref[...], 0.0)
        v = jnp.dot(h, w_ref[...], preferred_element_type=jnp.float32) * dis_v
        o_ref[0] = v[:, :64]
        o_ref[1] = v[:, 64:]

    return pl.pallas_call(
        body,
        grid=(GN,),
        in_specs=[
            pl.BlockSpec((2, BR, 64), lambda i: (0, i, 0)),
            pl.BlockSpec((2, BR, 64), lambda i: (0, i, 0)),
            pl.BlockSpec((1, 256), lambda i: (0, 0)),
            pl.BlockSpec((BR, 1), lambda i: (i, 0)),
            pl.BlockSpec((256, 128), lambda i: (0, 0)),
        ],
        out_specs=pl.BlockSpec((2, BR, 64), lambda i: (0, i, 0)),
        out_shape=jax.ShapeDtypeStruct((2, NP, 64), jnp.float32),
    )(pa, pb, b, dis, w)


def _tc_fin(p, bmu, bls, dis):
    def body(p_ref, bmu_ref, bls_ref, dis_ref, mu_ref, ls_ref):
        dis_v = dis_ref[...]
        mu_ref[...] = p_ref[0] * dis_v + bmu_ref[...]
        ls_ref[...] = p_ref[1] * dis_v + bls_ref[...]

    return pl.pallas_call(
        body,
        grid=(GN,),
        in_specs=[
            pl.BlockSpec((2, BR, 64), lambda i: (0, i, 0)),
            pl.BlockSpec((1, 64), lambda i: (0, 0)),
            pl.BlockSpec((1, 64), lambda i: (0, 0)),
            pl.BlockSpec((BR, 1), lambda i: (i, 0)),
        ],
        out_specs=[
            pl.BlockSpec((BR, 64), lambda i: (i, 0)),
            pl.BlockSpec((BR, 64), lambda i: (i, 0)),
        ],
        out_shape=[
            jax.ShapeDtypeStruct((NP, 64), jnp.float32),
            jax.ShapeDtypeStruct((NP, 64), jnp.float32),
        ],
    )(p, bmu, bls, dis)


def _tc_dot(zs, zd):
    def body(zs_ref, zd_ref, o_ref):
        v = jnp.sum(zs_ref[...] * zd_ref[...], axis=1, keepdims=True)
        o_ref[...] = 1.0 / (1.0 + jnp.exp(-v))

    return pl.pallas_call(
        body,
        grid=(GE,),
        in_specs=[
            pl.BlockSpec((BRE, 64), lambda i: (i, 0)),
            pl.BlockSpec((BRE, 64), lambda i: (i, 0)),
        ],
        out_specs=pl.BlockSpec((BRE, 1), lambda i: (i, 0)),
        out_shape=jax.ShapeDtypeStruct((EP, 1), jnp.float32),
    )(zs, zd)


def kernel(x, edge_index, W1, b1, W2, b2, Wmu, bmu, Wls, bls):
    xp = jnp.pad(x, ((0, NP - N), (0, 0)))
    pad = jnp.full((EP - E,), N, jnp.int32)
    srcp = jnp.concatenate([edge_index[0], pad]).reshape(ERP, K)
    dstp = jnp.concatenate([edge_index[1], pad]).reshape(ERP, K)
    w3 = jnp.concatenate([Wmu, Wls], axis=1)

    degp = _deg(dstp).reshape(2, NP, 1)
    dis, v1 = _tc_prep(xp, W1, degp)
    p1 = _agg(v1.reshape(2 * NP, 64), srcp, dstp)
    v2a, v2b = _tc_mid1(p1.reshape(2, NP, 64), b1.reshape(1, 128), dis, W2)
    p2a = _agg(v2a.reshape(2 * NP, 64), srcp, dstp)
    p2b = _agg(v2b.reshape(2 * NP, 64), srcp, dstp)
    v3 = _tc_mid2(p2a.reshape(2, NP, 64), p2b.reshape(2, NP, 64),
                  b2.reshape(1, 256), dis, w3)
    p3 = _agg(v3.reshape(2 * NP, 64), srcp, dstp)
    mu, logstd = _tc_fin(p3.reshape(2, NP, 64), bmu.reshape(1, 64),
                         bls.reshape(1, 64), dis)
    zs, zd = _dec(mu, srcp, dstp)
    recon = _tc_dot(zs, zd)
    return (recon.reshape(-1)[:E], mu[:N], logstd[:N])
